# baseline (speedup 1.0000x reference)
import jax
import jax.numpy as jnp
from jax import lax
from jax.experimental import pallas as pl
from jax.experimental.pallas import tpu as pltpu
from jax.experimental.pallas import tpu_sc as plsc

_N = 10000
_E = 320000
_D = 128
_NB = 64
_NC = 2
_NS = 16
_EC = _E // _NC
_ET = _EC // _NS
_CH = 80
_NCH = _ET // _CH
_NP = 10240
_RT = _NP // _NS
_WB = 128
_DEGW = _D

_BLK = 1000
_G = _N // _BLK


def _fill(ref, rows, width, value):
    v = jnp.full((16,), value, jnp.float32)

    def row(r, carry):
        def col(j, c2):
            ref[r, pl.ds(j * 16, 16)] = v
            return c2

        return lax.fori_loop(0, width // 16, col, carry)

    lax.fori_loop(0, rows, row, 0)


def _sc_agg(y, src, dst):
    mesh = plsc.VectorSubcoreMesh(core_axis_name="c", subcore_axis_name="s")

    def body(y_hbm, src_hbm, dst_hbm, part_hbm,
             src0, dst0, rows0, src1, dst1, rows1, bounce_v, acc, sem0, sem1):
        c = lax.axis_index("c")
        s = lax.axis_index("s")
        r0 = s * _RT

        _fill(bounce_v, _WB, _D, 0.0)
        for k in range(_RT // _WB):
            pltpu.sync_copy(bounce_v, acc.at[pl.ds(r0 + k * _WB, _WB)])
        plsc.subcore_barrier()

        ebase = c * _EC + s * _ET
        bufs = ((src0, dst0, rows0, sem0), (src1, dst1, rows1, sem1))

        def fire(b, i):
            sb, db, rb, sem = bufs[b]
            off = ebase + i * _CH
            pltpu.sync_copy(src_hbm.at[pl.ds(off, _CH)], sb.at[0])
            pltpu.sync_copy(dst_hbm.at[pl.ds(off, _CH)], db.at[0])
            pltpu.async_copy(y_hbm.at[sb.at[0]], rb, sem)

        def drain_scatter(b):
            sb, db, rb, sem = bufs[b]
            pltpu.make_async_copy(y_hbm.at[pl.ds(0, _CH)], rb, sem).wait()
            pltpu.sync_copy(rb, acc.at[db.at[0]], add=True)

        fire(0, 0)
        fire(1, 1)
        _G2 = (_NCH - 1) // 2

        def step(g, carry):
            drain_scatter(0)
            fire(0, 2 * g + 2)
            drain_scatter(1)

            @pl.when(g < _G2 - 1)
            def _():
                fire(1, 2 * g + 3)

            return carry

        lax.fori_loop(0, _G2, step, 0)
        drain_scatter(0)
        plsc.subcore_barrier()

        for k in range(_RT // _WB):
            pltpu.sync_copy(acc.at[pl.ds(r0 + k * _WB, _WB)], bounce_v)
            pltpu.sync_copy(bounce_v, part_hbm.at[c, pl.ds(r0 + k * _WB, _WB)])

    f = pl.kernel(
        body,
        out_type=jax.ShapeDtypeStruct((_NC, _NP, _D), jnp.float32),
        mesh=mesh,
        scratch_types=[
            pltpu.VMEM((1, _CH), jnp.int32),
            pltpu.VMEM((1, _CH), jnp.int32),
            pltpu.VMEM((_CH, _D), jnp.float32),
            pltpu.VMEM((1, _CH), jnp.int32),
            pltpu.VMEM((1, _CH), jnp.int32),
            pltpu.VMEM((_CH, _D), jnp.float32),
            pltpu.VMEM((_WB, _D), jnp.float32),
            pltpu.VMEM_SHARED((_NP, _D), jnp.float32),
            pltpu.SemaphoreType.DMA,
            pltpu.SemaphoreType.DMA,
        ],
    )
    return f(y, src, dst)


def _tc_front_body(x_ref, ws_ref, wn_ref, b_ref, u_ref, y_ref):
    h = x_ref[...]
    u_ref[...] = jnp.dot(h, ws_ref[...], preferred_element_type=jnp.float32) + b_ref[...]
    y_ref[...] = jnp.dot(h, wn_ref[...], preferred_element_type=jnp.float32)


def _tc_front(x, Ws, Wn, bsum):
    return pl.pallas_call(
        _tc_front_body,
        grid=(_G,),
        in_specs=[
            pl.BlockSpec((_BLK, _D), lambda i: (i, 0)),
            pl.BlockSpec((_D, _D), lambda i: (0, 0)),
            pl.BlockSpec((_D, _D), lambda i: (0, 0)),
            pl.BlockSpec((1, _D), lambda i: (0, 0)),
        ],
        out_specs=[
            pl.BlockSpec((_BLK, _D), lambda i: (i, 0)),
            pl.BlockSpec((_BLK, _D), lambda i: (i, 0)),
        ],
        out_shape=[jax.ShapeDtypeStruct((_N, _D), jnp.float32)] * 2,
    )(x, Ws, Wn, bsum)


def _relu_layer(u_ref, p_ref, degp_ref):
    agg = p_ref[0] + p_ref[1]
    deg = degp_ref[0, :, 0:1] + degp_ref[1, :, 0:1]
    return jnp.maximum(u_ref[...] + agg / jnp.maximum(deg, 1.0), 0.0)


def _tc_mid_body(u_ref, p_ref, degp_ref, ws_ref, wn_ref, b_ref, u2_ref, y2_ref):
    h = _relu_layer(u_ref, p_ref, degp_ref)
    u2_ref[...] = jnp.dot(h, ws_ref[...], preferred_element_type=jnp.float32) + b_ref[...]
    y2_ref[...] = jnp.dot(h, wn_ref[...], preferred_element_type=jnp.float32)


def _tc_mid(u, part, degp, Ws, Wn, bsum):
    return pl.pallas_call(
        _tc_mid_body,
        grid=(_G,),
        in_specs=[
            pl.BlockSpec((_BLK, _D), lambda i: (i, 0)),
            pl.BlockSpec((_NC, _BLK, _D), lambda i: (0, i, 0)),
            pl.BlockSpec((_NC, _BLK, _DEGW), lambda i: (0, i, 0)),
            pl.BlockSpec((_D, _D), lambda i: (0, 0)),
            pl.BlockSpec((_D, _D), lambda i: (0, 0)),
            pl.BlockSpec((1, _D), lambda i: (0, 0)),
        ],
        out_specs=[
            pl.BlockSpec((_BLK, _D), lambda i: (i, 0)),
            pl.BlockSpec((_BLK, _D), lambda i: (i, 0)),
        ],
        out_shape=[jax.ShapeDtypeStruct((_N, _D), jnp.float32)] * 2,
    )(u, part, degp, Ws, Wn, bsum)


def _tc_final_body(u_ref, p_ref, degp_ref, bv_ref, out_ref, acc_ref, cnt_ref):
    i = pl.program_id(0)

    @pl.when(i == 0)
    def _init():
        acc_ref[...] = jnp.zeros_like(acc_ref)
        cnt_ref[...] = jnp.zeros_like(cnt_ref)

    h = _relu_layer(u_ref, p_ref, degp_ref)
    bv = bv_ref[0, 0, :]
    sel = (lax.broadcasted_iota(jnp.int32, (_NB, _BLK), 0) == bv[None, :]).astype(jnp.float32)
    acc_ref[...] += jnp.dot(sel, h, preferred_element_type=jnp.float32)
    cnt_ref[...] += jnp.sum(sel, axis=1, keepdims=True)

    @pl.when(i == _G - 1)
    def _fin():
        out_ref[...] = acc_ref[...] / jnp.maximum(cnt_ref[...], 1.0)


def _tc_final(u, part, degp, bv):
    return pl.pallas_call(
        _tc_final_body,
        grid=(_G,),
        in_specs=[
            pl.BlockSpec((_BLK, _D), lambda i: (i, 0)),
            pl.BlockSpec((_NC, _BLK, _D), lambda i: (0, i, 0)),
            pl.BlockSpec((_NC, _BLK, _DEGW), lambda i: (0, i, 0)),
            pl.BlockSpec((1, 1, _BLK), lambda i: (i, 0, 0)),
        ],
        out_specs=pl.BlockSpec((_NB, _D), lambda i: (0, 0)),
        out_shape=jax.ShapeDtypeStruct((_NB, _D), jnp.float32),
        scratch_shapes=[
            pltpu.VMEM((_NB, _D), jnp.float32),
            pltpu.VMEM((_NB, _D), jnp.float32),
        ],
    )(u, part, degp, bv)


def kernel(x, edge_index, batch_vec, Ws0, bs0, Wn0, bn0,
           Ws1, bs1, Wn1, bn1, Ws2, bs2, Wn2, bn2):
    b0 = (bs0 + bn0).reshape(1, _D)
    b1 = (bs1 + bn1).reshape(1, _D)
    b2 = (bs2 + bn2).reshape(1, _D)
    bv = batch_vec.reshape(_G, 1, _BLK)
    src = edge_index[0]
    dst = edge_index[1]

    ones = jnp.ones((_N, _D), jnp.float32)
    degp = _sc_agg(ones, jnp.zeros((_E,), jnp.int32), dst)
    u0, y0 = _tc_front(x, Ws0, Wn0, b0)
    p0 = _sc_agg(y0, src, dst)
    u1, y1 = _tc_mid(u0, p0, degp, Ws1, Wn1, b1)
    p1 = _sc_agg(y1, src, dst)
    u2, y2 = _tc_mid(u1, p1, degp, Ws2, Wn2, b2)
    p2 = _sc_agg(y2, src, dst)
    return _tc_final(u2, p2, degp, bv)


# speedup vs baseline: 15.9805x; 15.9805x over previous
import jax
import jax.numpy as jnp
from jax import lax
from jax.experimental import pallas as pl
from jax.experimental.pallas import tpu as pltpu
from jax.experimental.pallas import tpu_sc as plsc

_N = 10000
_E = 320000
_D = 128
_NB = 64
_NC = 2
_NS = 16
_EC = _E // _NC
_ET = _EC // _NS
_CH = 80
_NCH = _ET // _CH
_NBUF = 3
_NP = 10240
_RT = _NP // _NS
_WB = 64

_BLK = 1000
_G = _N // _BLK


def _fill(ref, rows, width, value):
    v = jnp.full((16,), value, jnp.float32)

    def row(r, carry):
        def col(j, c2):
            ref[r, pl.ds(j * 16, 16)] = v
            return c2

        return lax.fori_loop(0, width // 16, col, carry)

    lax.fori_loop(0, rows, row, 0)


def _sc_agg(y, src, dst):
    mesh = plsc.VectorSubcoreMesh(core_axis_name="c", subcore_axis_name="s")

    def body(y_hbm, src_hbm, dst_hbm, part_hbm, *sc):
        bufs = tuple(zip(sc[0:3 * _NBUF:3],
                         sc[1:3 * _NBUF:3],
                         sc[2:3 * _NBUF:3],
                         sc[3 * _NBUF + 2:]))
        bounce_v = sc[3 * _NBUF]
        acc = sc[3 * _NBUF + 1]

        c = lax.axis_index("c")
        s = lax.axis_index("s")
        r0 = s * _RT

        _fill(bounce_v, _WB, _D, 0.0)
        for k in range(_RT // _WB):
            pltpu.sync_copy(bounce_v, acc.at[pl.ds(r0 + k * _WB, _WB)])
        plsc.subcore_barrier()

        ebase = c * _EC + s * _ET

        def fire(b, i):
            sb, db, rb, sem = bufs[b]
            off = ebase + i * _CH
            pltpu.sync_copy(src_hbm.at[pl.ds(off, _CH)], sb.at[0])
            pltpu.sync_copy(dst_hbm.at[pl.ds(off, _CH)], db.at[0])
            pltpu.async_copy(y_hbm.at[sb.at[0]], rb, sem)

        def drain_scatter(b):
            sb, db, rb, sem = bufs[b]
            pltpu.make_async_copy(y_hbm.at[pl.ds(0, _CH)], rb, sem).wait()
            pltpu.sync_copy(rb, acc.at[db.at[0]], add=True)

        for b in range(_NBUF):
            fire(b, b)

        def step(g, carry):
            for b in range(_NBUF):
                i = g * _NBUF + b

                @pl.when(i < _NCH)
                def _():
                    drain_scatter(b)

                    @pl.when(i + _NBUF < _NCH)
                    def _():
                        fire(b, i + _NBUF)

            return carry

        lax.fori_loop(0, (_NCH + _NBUF - 1) // _NBUF, step, 0)
        plsc.subcore_barrier()

        for k in range(_RT // _WB):
            pltpu.sync_copy(acc.at[pl.ds(r0 + k * _WB, _WB)], bounce_v)
            pltpu.sync_copy(bounce_v, part_hbm.at[c, pl.ds(r0 + k * _WB, _WB)])

    ring = []
    for _ in range(_NBUF):
        ring += [
            pltpu.VMEM((1, _CH), jnp.int32),
            pltpu.VMEM((1, _CH), jnp.int32),
            pltpu.VMEM((_CH, _D), jnp.float32),
        ]
    f = pl.kernel(
        body,
        out_type=jax.ShapeDtypeStruct((_NC, _NP, _D), jnp.float32),
        mesh=mesh,
        scratch_types=ring + [
            pltpu.VMEM((_WB, _D), jnp.float32),
            pltpu.VMEM_SHARED((_NP, _D), jnp.float32),
        ] + [pltpu.SemaphoreType.DMA] * _NBUF,
    )
    return f(y, src, dst)


def _tc_front_body(x_ref, ws_ref, wn_ref, b_ref, u_ref, y_ref):
    h = x_ref[...]
    u_ref[...] = jnp.dot(h, ws_ref[...], preferred_element_type=jnp.float32) + b_ref[...]
    y_ref[...] = jnp.dot(h, wn_ref[...], preferred_element_type=jnp.float32)


def _tc_front(x, Ws, Wn, bsum):
    return pl.pallas_call(
        _tc_front_body,
        grid=(_G,),
        in_specs=[
            pl.BlockSpec((_BLK, _D), lambda i: (i, 0)),
            pl.BlockSpec((_D, _D), lambda i: (0, 0)),
            pl.BlockSpec((_D, _D), lambda i: (0, 0)),
            pl.BlockSpec((1, _D), lambda i: (0, 0)),
        ],
        out_specs=[
            pl.BlockSpec((_BLK, _D), lambda i: (i, 0)),
            pl.BlockSpec((_BLK, _D), lambda i: (i, 0)),
        ],
        out_shape=[jax.ShapeDtypeStruct((_N, _D), jnp.float32)] * 2,
    )(x, Ws, Wn, bsum)


def _relu_layer(u_ref, p_ref, degp_ref):
    agg = p_ref[0] + p_ref[1]
    deg = degp_ref[0, :, 0:1] + degp_ref[1, :, 0:1]
    return jnp.maximum(u_ref[...] + agg / jnp.maximum(deg, 1.0), 0.0)


def _tc_mid_body(u_ref, p_ref, degp_ref, ws_ref, wn_ref, b_ref, u2_ref, y2_ref):
    h = _relu_layer(u_ref, p_ref, degp_ref)
    u2_ref[...] = jnp.dot(h, ws_ref[...], preferred_element_type=jnp.float32) + b_ref[...]
    y2_ref[...] = jnp.dot(h, wn_ref[...], preferred_element_type=jnp.float32)


def _tc_mid(u, part, degp, Ws, Wn, bsum):
    return pl.pallas_call(
        _tc_mid_body,
        grid=(_G,),
        in_specs=[
            pl.BlockSpec((_BLK, _D), lambda i: (i, 0)),
            pl.BlockSpec((_NC, _BLK, _D), lambda i: (0, i, 0)),
            pl.BlockSpec((_NC, _BLK, _D), lambda i: (0, i, 0)),
            pl.BlockSpec((_D, _D), lambda i: (0, 0)),
            pl.BlockSpec((_D, _D), lambda i: (0, 0)),
            pl.BlockSpec((1, _D), lambda i: (0, 0)),
        ],
        out_specs=[
            pl.BlockSpec((_BLK, _D), lambda i: (i, 0)),
            pl.BlockSpec((_BLK, _D), lambda i: (i, 0)),
        ],
        out_shape=[jax.ShapeDtypeStruct((_N, _D), jnp.float32)] * 2,
    )(u, part, degp, Ws, Wn, bsum)


def _tc_final_body(u_ref, p_ref, degp_ref, bv_ref, out_ref, acc_ref, cnt_ref):
    i = pl.program_id(0)

    @pl.when(i == 0)
    def _init():
        acc_ref[...] = jnp.zeros_like(acc_ref)
        cnt_ref[...] = jnp.zeros_like(cnt_ref)

    h = _relu_layer(u_ref, p_ref, degp_ref)
    bv = bv_ref[0, 0, :]
    sel = (lax.broadcasted_iota(jnp.int32, (_NB, _BLK), 0) == bv[None, :]).astype(jnp.float32)
    acc_ref[...] += jnp.dot(sel, h, preferred_element_type=jnp.float32)
    cnt_ref[...] += jnp.sum(sel, axis=1, keepdims=True)

    @pl.when(i == _G - 1)
    def _fin():
        out_ref[...] = acc_ref[...] / jnp.maximum(cnt_ref[...], 1.0)


def _tc_final(u, part, degp, bv):
    return pl.pallas_call(
        _tc_final_body,
        grid=(_G,),
        in_specs=[
            pl.BlockSpec((_BLK, _D), lambda i: (i, 0)),
            pl.BlockSpec((_NC, _BLK, _D), lambda i: (0, i, 0)),
            pl.BlockSpec((_NC, _BLK, _D), lambda i: (0, i, 0)),
            pl.BlockSpec((1, 1, _BLK), lambda i: (i, 0, 0)),
        ],
        out_specs=pl.BlockSpec((_NB, _D), lambda i: (0, 0)),
        out_shape=jax.ShapeDtypeStruct((_NB, _D), jnp.float32),
        scratch_shapes=[
            pltpu.VMEM((_NB, _D), jnp.float32),
            pltpu.VMEM((_NB, _D), jnp.float32),
        ],
    )(u, part, degp, bv)


def kernel(x, edge_index, batch_vec, Ws0, bs0, Wn0, bn0,
           Ws1, bs1, Wn1, bn1, Ws2, bs2, Wn2, bn2):
    b0 = (bs0 + bn0).reshape(1, _D)
    b1 = (bs1 + bn1).reshape(1, _D)
    b2 = (bs2 + bn2).reshape(1, _D)
    bv = batch_vec.reshape(_G, 1, _BLK)
    src = edge_index[0]
    dst = edge_index[1]

    ones = jnp.ones((_N, _D), jnp.float32)
    degp = _sc_agg(ones, src, dst)
    u0, y0 = _tc_front(x, Ws0, Wn0, b0)
    p0 = _sc_agg(y0, src, dst)
    u1, y1 = _tc_mid(u0, p0, degp, Ws1, Wn1, b1)
    p1 = _sc_agg(y1, src, dst)
    u2, y2 = _tc_mid(u1, p1, degp, Ws2, Wn2, b2)
    p2 = _sc_agg(y2, src, dst)
    return _tc_final(u2, p2, degp, bv)


# speedup vs baseline: 22.1669x; 1.3871x over previous
import jax
import jax.numpy as jnp
from jax import lax
from jax.experimental import pallas as pl
from jax.experimental.pallas import tpu as pltpu
from jax.experimental.pallas import tpu_sc as plsc

_N = 10000
_E = 320000
_D = 128
_NB = 64
_NC = 2
_NS = 16
_EC = _E // _NC
_ET = _EC // _NS
_CH = 80
_NCH = _ET // _CH
_NBUF = 2
_NP = 10240
_RT = _NP // _NS
_WB = 16

_BLK = 1000
_G = _N // _BLK


def _fill(ref, rows, width, value):
    v = jnp.full((16,), value, jnp.float32)

    def row(r, carry):
        def col(j, c2):
            ref[r, pl.ds(j * 16, 16)] = v
            return c2

        return lax.fori_loop(0, width // 16, col, carry)

    lax.fori_loop(0, rows, row, 0)


def _sc_agg(y, src, dst):
    mesh = plsc.VectorSubcoreMesh(core_axis_name="c", subcore_axis_name="s")

    def body(y_hbm, src_hbm, dst_hbm, part_hbm, *sc):
        bufs = tuple(zip(sc[0:2 * _NBUF:2],
                         sc[1:2 * _NBUF:2],
                         sc[2 * _NBUF + 3:2 * _NBUF + 3 + _NBUF],
                         sc[2 * _NBUF + 3 + _NBUF:]))
        src_all = sc[2 * _NBUF]
        bounce_v = sc[2 * _NBUF + 1]
        acc = sc[2 * _NBUF + 2]

        c = lax.axis_index("c")
        s = lax.axis_index("s")
        r0 = s * _RT

        pltpu.sync_copy(src_hbm.at[c, s], src_all)

        _fill(bounce_v, _WB, _D, 0.0)
        for k in range(_RT // _WB):
            pltpu.sync_copy(bounce_v, acc.at[pl.ds(r0 + k * _WB, _WB)])
        plsc.subcore_barrier()

        def fire(b, i):
            db, rb, semg, semd = bufs[b]
            pltpu.async_copy(dst_hbm.at[c, s, i], db.at[0], semd)
            pltpu.async_copy(y_hbm.at[src_all.at[i]], rb, semg)

        def drain_scatter(b, i):
            db, rb, semg, semd = bufs[b]
            pltpu.make_async_copy(dst_hbm.at[0, 0, 0], db.at[0], semd).wait()
            pltpu.make_async_copy(y_hbm.at[pl.ds(0, _CH)], rb, semg).wait()
            pltpu.sync_copy(rb, acc.at[db.at[0]], add=True)

        for b in range(_NBUF):
            fire(b, b)

        def step(g, carry):
            for b in range(_NBUF):
                i = g * _NBUF + b

                @pl.when(i < _NCH)
                def _():
                    drain_scatter(b, i)

                    @pl.when(i + _NBUF < _NCH)
                    def _():
                        fire(b, i + _NBUF)

            return carry

        lax.fori_loop(0, (_NCH + _NBUF - 1) // _NBUF, step, 0)
        plsc.subcore_barrier()

        for k in range(_RT // _WB):
            pltpu.sync_copy(acc.at[pl.ds(r0 + k * _WB, _WB)], bounce_v)
            pltpu.sync_copy(bounce_v, part_hbm.at[c, pl.ds(r0 + k * _WB, _WB)])

    ring = []
    for _ in range(_NBUF):
        ring += [
            pltpu.VMEM((1, _CH), jnp.int32),
            pltpu.VMEM((_CH, _D), jnp.float32),
        ]
    f = pl.kernel(
        body,
        out_type=jax.ShapeDtypeStruct((_NC, _NP, _D), jnp.float32),
        mesh=mesh,
        scratch_types=ring + [
            pltpu.VMEM((_NCH, _CH), jnp.int32),
            pltpu.VMEM((_WB, _D), jnp.float32),
            pltpu.VMEM_SHARED((_NP, _D), jnp.float32),
        ] + [pltpu.SemaphoreType.DMA] * (2 * _NBUF),
    )
    return f(y, src, dst)


def _tc_front_body(x_ref, ws_ref, wn_ref, b_ref, u_ref, y_ref):
    h = x_ref[...]
    u_ref[...] = jnp.dot(h, ws_ref[...], preferred_element_type=jnp.float32) + b_ref[...]
    y_ref[...] = jnp.dot(h, wn_ref[...], preferred_element_type=jnp.float32)


def _tc_front(x, Ws, Wn, bsum):
    return pl.pallas_call(
        _tc_front_body,
        grid=(_G,),
        in_specs=[
            pl.BlockSpec((_BLK, _D), lambda i: (i, 0)),
            pl.BlockSpec((_D, _D), lambda i: (0, 0)),
            pl.BlockSpec((_D, _D), lambda i: (0, 0)),
            pl.BlockSpec((1, _D), lambda i: (0, 0)),
        ],
        out_specs=[
            pl.BlockSpec((_BLK, _D), lambda i: (i, 0)),
            pl.BlockSpec((_BLK, _D), lambda i: (i, 0)),
        ],
        out_shape=[jax.ShapeDtypeStruct((_N, _D), jnp.float32)] * 2,
    )(x, Ws, Wn, bsum)


def _relu_layer(u_ref, p_ref, degp_ref):
    agg = p_ref[0] + p_ref[1]
    deg = degp_ref[0, :, 0:1] + degp_ref[1, :, 0:1]
    return jnp.maximum(u_ref[...] + agg / jnp.maximum(deg, 1.0), 0.0)


def _tc_mid_body(u_ref, p_ref, degp_ref, ws_ref, wn_ref, b_ref, u2_ref, y2_ref):
    h = _relu_layer(u_ref, p_ref, degp_ref)
    u2_ref[...] = jnp.dot(h, ws_ref[...], preferred_element_type=jnp.float32) + b_ref[...]
    y2_ref[...] = jnp.dot(h, wn_ref[...], preferred_element_type=jnp.float32)


def _tc_mid(u, part, degp, Ws, Wn, bsum):
    return pl.pallas_call(
        _tc_mid_body,
        grid=(_G,),
        in_specs=[
            pl.BlockSpec((_BLK, _D), lambda i: (i, 0)),
            pl.BlockSpec((_NC, _BLK, _D), lambda i: (0, i, 0)),
            pl.BlockSpec((_NC, _BLK, _D), lambda i: (0, i, 0)),
            pl.BlockSpec((_D, _D), lambda i: (0, 0)),
            pl.BlockSpec((_D, _D), lambda i: (0, 0)),
            pl.BlockSpec((1, _D), lambda i: (0, 0)),
        ],
        out_specs=[
            pl.BlockSpec((_BLK, _D), lambda i: (i, 0)),
            pl.BlockSpec((_BLK, _D), lambda i: (i, 0)),
        ],
        out_shape=[jax.ShapeDtypeStruct((_N, _D), jnp.float32)] * 2,
    )(u, part, degp, Ws, Wn, bsum)


def _tc_final_body(u_ref, p_ref, degp_ref, bv_ref, out_ref, acc_ref, cnt_ref):
    i = pl.program_id(0)

    @pl.when(i == 0)
    def _init():
        acc_ref[...] = jnp.zeros_like(acc_ref)
        cnt_ref[...] = jnp.zeros_like(cnt_ref)

    h = _relu_layer(u_ref, p_ref, degp_ref)
    bv = bv_ref[0, 0, :]
    sel = (lax.broadcasted_iota(jnp.int32, (_NB, _BLK), 0) == bv[None, :]).astype(jnp.float32)
    acc_ref[...] += jnp.dot(sel, h, preferred_element_type=jnp.float32)
    cnt_ref[...] += jnp.sum(sel, axis=1, keepdims=True)

    @pl.when(i == _G - 1)
    def _fin():
        out_ref[...] = acc_ref[...] / jnp.maximum(cnt_ref[...], 1.0)


def _tc_final(u, part, degp, bv):
    return pl.pallas_call(
        _tc_final_body,
        grid=(_G,),
        in_specs=[
            pl.BlockSpec((_BLK, _D), lambda i: (i, 0)),
            pl.BlockSpec((_NC, _BLK, _D), lambda i: (0, i, 0)),
            pl.BlockSpec((_NC, _BLK, _D), lambda i: (0, i, 0)),
            pl.BlockSpec((1, 1, _BLK), lambda i: (i, 0, 0)),
        ],
        out_specs=pl.BlockSpec((_NB, _D), lambda i: (0, 0)),
        out_shape=jax.ShapeDtypeStruct((_NB, _D), jnp.float32),
        scratch_shapes=[
            pltpu.VMEM((_NB, _D), jnp.float32),
            pltpu.VMEM((_NB, _D), jnp.float32),
        ],
    )(u, part, degp, bv)


def kernel(x, edge_index, batch_vec, Ws0, bs0, Wn0, bn0,
           Ws1, bs1, Wn1, bn1, Ws2, bs2, Wn2, bn2):
    b0 = (bs0 + bn0).reshape(1, _D)
    b1 = (bs1 + bn1).reshape(1, _D)
    b2 = (bs2 + bn2).reshape(1, _D)
    bv = batch_vec.reshape(_G, 1, _BLK)
    src = edge_index[0]
    dst = edge_index[1]

    ones = jnp.ones((_N, _D), jnp.float32)
    src = src.reshape(_NC, _NS, _NCH, _CH)
    dst = dst.reshape(_NC, _NS, _NCH, _CH)
    degp = _sc_agg(ones, src, dst)
    u0, y0 = _tc_front(x, Ws0, Wn0, b0)
    p0 = _sc_agg(y0, src, dst)
    u1, y1 = _tc_mid(u0, p0, degp, Ws1, Wn1, b1)
    p1 = _sc_agg(y1, src, dst)
    u2, y2 = _tc_mid(u1, p1, degp, Ws2, Wn2, b2)
    p2 = _sc_agg(y2, src, dst)
    return _tc_final(u2, p2, degp, bv)
